# baseline (speedup 1.0000x reference)
import functools

import jax
import jax.numpy as jnp
from jax import lax
from jax.experimental import pallas as pl
from jax.experimental.pallas import tpu as pltpu
from jax.experimental.pallas import tpu_sc as plsc

_NA = 50000
_E = 1600000
_NMAX = 2
_NO = 4
_RCUT = 6.0

_NW = 32
_LANES = 16
_G = 4
_SB = 25
_A_TILE = _G * _LANES * _SB
_NA_PAD = _A_TILE * _NW
_NGROUP = _NA_PAD // _G
_B = 2560
_ASTRIDE = 9
_ACC_W = _A_TILE * _ASTRIDE
_RHO_W = _A_TILE * _NO
_S4_W = 12808

_COS_C = (
    0.999999999798843,
    -0.49999999819665425,
    0.041666663366875245,
    -0.001388886287562638,
    2.4800552409794075e-05,
    -2.753479909104947e-07,
    2.0603598271939916e-09,
    -9.722556093549883e-12,
)
_PI = 3.14159265358979323846


def _sc_body(disp_h, ii_h, jj_h, sn_h, consts_h, s4_h, out_h,
             sn_v, consts_v, s4_v, dsp_v, ii_v, jj_v, acc, rho_v):
    wid = lax.axis_index("s") * 2 + lax.axis_index("c")
    abase = wid * _A_TILE

    pltpu.sync_copy(sn_h, sn_v)
    pltpu.sync_copy(consts_h, consts_v)
    pltpu.sync_copy(s4_h, s4_v)

    iota = lax.iota(jnp.int32, _LANES)

    def _zero_acc(i, c):
        acc[pl.ds(i * _LANES, _LANES)] = jnp.zeros((_LANES,), jnp.float32)
        return c
    lax.fori_loop(0, _ACC_W // _LANES, _zero_acc, 0)

    def _superblock(g, c0):
        gbase = wid * (_SB * _LANES) + g * _LANES
        Lv = plsc.load_gather(s4_v, [gbase + iota])
        Hv = plsc.load_gather(s4_v, [gbase + 1 + iota])
        lo = jnp.min(Lv)
        hi = jnp.max(Hv)
        lo8 = lax.bitwise_and(lo, jnp.int32(-8))
        nchunks = (hi - lo8 + _B - 1) // _B

        def _chunk(b, c1):
            b0 = lo8 + b * _B
            b0 = pl.multiple_of(jnp.minimum(b0, _E - _B), 8)
            pltpu.sync_copy(disp_h.at[pl.ds(b0 * 3, _B * 3)], dsp_v)
            pltpu.sync_copy(ii_h.at[pl.ds(b0, _B)], ii_v)
            pltpu.sync_copy(jj_h.at[pl.ds(b0, _B)], jj_v)

            blo = jnp.maximum(Lv, b0)
            bhi = jnp.minimum(Hv, b0 + _B)
            cnt = bhi - blo
            m = jnp.max(cnt)
            base0 = blo - b0

            def _one(j):
                act = j < cnt
                eidx = jnp.where(act, base0 + j, 0)
                e3 = eidx * 3
                ddx = plsc.load_gather(dsp_v, [e3])
                ddy = plsc.load_gather(dsp_v, [e3 + 1])
                ddz = plsc.load_gather(dsp_v, [e3 + 2])
                ii = plsc.load_gather(ii_v, [eidx])
                jj = plsc.load_gather(jj_v, [eidx])
                jat = plsc.load_gather(sn_v, [jj])

                s2 = ddx * ddx + ddy * ddy + ddz * ddz
                s2 = jnp.maximum(s2, 1e-12)
                yi = jnp.int32(0x5F3759DF) - lax.shift_right_logical(
                    plsc.bitcast(s2, jnp.int32), jnp.int32(1))
                y = plsc.bitcast(yi, jnp.float32)
                h = 0.5 * s2
                y = y * (1.5 - h * y * y)
                y = y * (1.5 - h * y * y)
                y = y * (1.5 - h * y * y)
                dist = s2 * y

                t = dist * (_PI / (2.0 * _RCUT))
                r = t * (1.0 / (2.0 * _PI))
                rk = (r + 12582912.0) - 12582912.0
                th = t - rk * (2.0 * _PI)
                x = th * th
                cpoly = _COS_C[7]
                for k in (6, 5, 4, 3, 2, 1, 0):
                    cpoly = cpoly * x + _COS_C[k]
                c2sq = cpoly * cpoly
                fc = c2sq * c2sq

                i2 = jat * 2
                a0 = plsc.load_gather(consts_v, [i2])
                a1 = plsc.load_gather(consts_v, [i2 + 1])
                r0 = plsc.load_gather(consts_v, [i2 + 6])
                r1 = plsc.load_gather(consts_v, [i2 + 7])
                sp0 = plsc.load_gather(consts_v, [i2 + 12])
                sp1 = plsc.load_gather(consts_v, [i2 + 13])

                u0 = dist - r0
                u1 = dist - r1
                g0 = fc * sp0 * jnp.exp(a0 * u0 * u0)
                g1 = fc * sp1 * jnp.exp(a1 * u1 * u1)

                base_i = jnp.where(act, (ii - abase) * _ASTRIDE, 0)
                plsc.addupdate_scatter(acc, [base_i], g0, mask=act)
                plsc.addupdate_scatter(acc, [base_i + 1], g1, mask=act)
                plsc.addupdate_scatter(acc, [base_i + 2], ddx * g0, mask=act)
                plsc.addupdate_scatter(acc, [base_i + 3], ddx * g1, mask=act)
                plsc.addupdate_scatter(acc, [base_i + 4], ddy * g0, mask=act)
                plsc.addupdate_scatter(acc, [base_i + 5], ddy * g1, mask=act)
                plsc.addupdate_scatter(acc, [base_i + 6], ddz * g0, mask=act)
                plsc.addupdate_scatter(acc, [base_i + 7], ddz * g1, mask=act)

            def _edge2(jj2, c2):
                j = jj2 * 2
                _one(j)
                _one(j + 1)
                return c2

            lax.fori_loop(0, (m + 1) // 2, _edge2, 0)
            return c1

        lax.fori_loop(0, nchunks, _chunk, 0)
        return c0

    lax.fori_loop(0, _SB, _superblock, 0)

    wv = []
    for l in range(_NO):
        for n in range(_NMAX):
            for o in range(_NO):
                j = 18 + l * 8 + n * 4 + o
                wv.append(plsc.load_gather(
                    consts_v, [jnp.full((_LANES,), j, jnp.int32)]))

    def _atoms(av, c):
        base = av * (_LANES * _ASTRIDE)
        bvec = []
        for ch in range(8):
            bvec.append(plsc.load_gather(acc, [base + iota * _ASTRIDE + ch]))
        for o in range(_NO):
            ro = None
            for l in range(_NO):
                tl = (wv[(l * 2) * 4 + o] * bvec[l * 2]
                      + wv[(l * 2 + 1) * 4 + o] * bvec[l * 2 + 1])
                ro = tl * tl if ro is None else ro + tl * tl
            plsc.store_scatter(rho_v, [av * (_LANES * 4) + iota * 4 + o], ro)
        return c

    lax.fori_loop(0, _A_TILE // _LANES, _atoms, 0)

    pltpu.sync_copy(rho_v, out_h.at[pl.ds(pl.multiple_of(wid * _RHO_W, 8),
                                          _RHO_W)])


def kernel(disp, iidx, jidx, sorted_numbers, alpha, rs, species_params,
           orbital_params):
    disp = disp.astype(jnp.float32)
    iidx = iidx.astype(jnp.int32)
    jidx = jidx.astype(jnp.int32)
    sorted_numbers = sorted_numbers.astype(jnp.int32)

    ge = lax.shift_right_logical(iidx, 2)
    counts = jnp.zeros((_NGROUP,), jnp.int32).at[ge].add(
        1, mode="drop", indices_are_sorted=True)
    s4 = jnp.concatenate([jnp.zeros((1,), jnp.int32),
                          jnp.cumsum(counts, dtype=jnp.int32)])
    s4 = jnp.concatenate(
        [s4, jnp.full((_S4_W - s4.shape[0],), _E, jnp.int32)])

    oidx = jnp.array([0, 1, 1, 1], dtype=jnp.int32)
    wln = orbital_params[0].astype(jnp.float32)[oidx]
    consts = jnp.concatenate([
        alpha.astype(jnp.float32).reshape(-1),
        rs.astype(jnp.float32).reshape(-1),
        species_params.astype(jnp.float32).reshape(-1),
        wln.reshape(-1),
        jnp.zeros((14,), jnp.float32),
    ])

    mesh = plsc.VectorSubcoreMesh(core_axis_name="c", subcore_axis_name="s")
    call = functools.partial(
        pl.kernel,
        mesh=mesh,
        compiler_params=pltpu.CompilerParams(needs_layout_passes=False),
        out_type=jax.ShapeDtypeStruct((_NA_PAD * _NO,), jnp.float32),
        scratch_types=[
            pltpu.VMEM((_NA,), jnp.int32),
            pltpu.VMEM((64,), jnp.float32),
            pltpu.VMEM((_S4_W,), jnp.int32),
            pltpu.VMEM((_B * 3,), jnp.float32),
            pltpu.VMEM((_B,), jnp.int32),
            pltpu.VMEM((_B,), jnp.int32),
            pltpu.VMEM((_ACC_W,), jnp.float32),
            pltpu.VMEM((_RHO_W,), jnp.float32),
        ],
    )(_sc_body)

    out = call(disp.reshape(-1), iidx, jidx, sorted_numbers, consts, s4)
    rho = out.reshape(_NA_PAD, _NO)[:_NA]
    return rho[None]


# speedup vs baseline: 9.6095x; 9.6095x over previous
import functools

import jax
import jax.numpy as jnp
from jax import lax
from jax.experimental import pallas as pl
from jax.experimental.pallas import tpu as pltpu
from jax.experimental.pallas import tpu_sc as plsc

_NA = 50000
_E = 1600000
_NMAX = 2
_NO = 4
_RCUT = 6.0

_NW = 32
_LANES = 16
_G = 4
_SB = 25
_A_TILE = _G * _LANES * _SB
_NA_PAD = _A_TILE * _NW
_NGROUP = _NA_PAD // _G
_B = 2560
_ASTRIDE = 9
_ACC_W = _A_TILE * _ASTRIDE
_RHO_W = _A_TILE * _NO
_S4_W = 12808

_COS_C = (
    0.999999999798843,
    -0.49999999819665425,
    0.041666663366875245,
    -0.001388886287562638,
    2.4800552409794075e-05,
    -2.753479909104947e-07,
    2.0603598271939916e-09,
    -9.722556093549883e-12,
)
_PI = 3.14159265358979323846


def _sc_body(dx_h, dy_h, dz_h, ii_h, jj_h, sn_h, consts_h, s4_h, out_h,
             sn_v, consts_v, s4_v, dx_v, dy_v, dz_v, ii_v, jj_v,
             acc, rho_v):
    wid = lax.axis_index("s") * 2 + lax.axis_index("c")
    abase = wid * _A_TILE

    pltpu.sync_copy(sn_h, sn_v)
    pltpu.sync_copy(consts_h, consts_v)
    pltpu.sync_copy(s4_h, s4_v)

    iota = lax.iota(jnp.int32, _LANES)

    def _zero_acc(i, c):
        acc[pl.ds(i * _LANES, _LANES)] = jnp.zeros((_LANES,), jnp.float32)
        return c
    lax.fori_loop(0, _ACC_W // _LANES, _zero_acc, 0)

    def _superblock(g, c0):
        gbase = wid * (_SB * _LANES) + g * _LANES
        Lv = plsc.load_gather(s4_v, [gbase + iota])
        Hv = plsc.load_gather(s4_v, [gbase + 1 + iota])
        lo = jnp.min(Lv)
        hi = jnp.max(Hv)
        lo8 = lax.bitwise_and(lo, jnp.int32(-8))
        nchunks = (hi - lo8 + _B - 1) // _B

        def _chunk(b, c1):
            b0 = lo8 + b * _B
            b0 = pl.multiple_of(jnp.minimum(b0, _E - _B), 8)
            pltpu.sync_copy(dx_h.at[pl.ds(b0, _B)], dx_v)
            pltpu.sync_copy(dy_h.at[pl.ds(b0, _B)], dy_v)
            pltpu.sync_copy(dz_h.at[pl.ds(b0, _B)], dz_v)
            pltpu.sync_copy(ii_h.at[pl.ds(b0, _B)], ii_v)
            pltpu.sync_copy(jj_h.at[pl.ds(b0, _B)], jj_v)

            blo = jnp.maximum(Lv, b0)
            bhi = jnp.minimum(Hv, b0 + _B)
            cnt = bhi - blo
            m = jnp.max(cnt)
            base0 = blo - b0

            def _one(j):
                act = j < cnt
                eidx = jnp.where(act, base0 + j, 0)
                ddx = plsc.load_gather(dx_v, [eidx])
                ddy = plsc.load_gather(dy_v, [eidx])
                ddz = plsc.load_gather(dz_v, [eidx])
                ii = plsc.load_gather(ii_v, [eidx])
                jj = plsc.load_gather(jj_v, [eidx])
                jat = plsc.load_gather(sn_v, [jj])

                s2 = ddx * ddx + ddy * ddy + ddz * ddz
                s2 = jnp.maximum(s2, 1e-12)
                yi = jnp.int32(0x5F3759DF) - lax.shift_right_logical(
                    plsc.bitcast(s2, jnp.int32), jnp.int32(1))
                y = plsc.bitcast(yi, jnp.float32)
                h = 0.5 * s2
                y = y * (1.5 - h * y * y)
                y = y * (1.5 - h * y * y)
                y = y * (1.5 - h * y * y)
                dist = s2 * y

                t = dist * (_PI / (2.0 * _RCUT))
                r = t * (1.0 / (2.0 * _PI))
                rk = (r + 12582912.0) - 12582912.0
                th = t - rk * (2.0 * _PI)
                x = th * th
                cpoly = _COS_C[7]
                for k in (6, 5, 4, 3, 2, 1, 0):
                    cpoly = cpoly * x + _COS_C[k]
                c2sq = cpoly * cpoly
                fc = c2sq * c2sq

                i2 = jat * 2
                a0 = plsc.load_gather(consts_v, [i2])
                a1 = plsc.load_gather(consts_v, [i2 + 1])
                r0 = plsc.load_gather(consts_v, [i2 + 6])
                r1 = plsc.load_gather(consts_v, [i2 + 7])
                sp0 = plsc.load_gather(consts_v, [i2 + 12])
                sp1 = plsc.load_gather(consts_v, [i2 + 13])

                u0 = dist - r0
                u1 = dist - r1
                g0 = fc * sp0 * jnp.exp(a0 * u0 * u0)
                g1 = fc * sp1 * jnp.exp(a1 * u1 * u1)

                base_i = jnp.where(act, (ii - abase) * _ASTRIDE, 0)
                plsc.addupdate_scatter(acc, [base_i], g0, mask=act)
                plsc.addupdate_scatter(acc, [base_i + 1], g1, mask=act)
                plsc.addupdate_scatter(acc, [base_i + 2], ddx * g0, mask=act)
                plsc.addupdate_scatter(acc, [base_i + 3], ddx * g1, mask=act)
                plsc.addupdate_scatter(acc, [base_i + 4], ddy * g0, mask=act)
                plsc.addupdate_scatter(acc, [base_i + 5], ddy * g1, mask=act)
                plsc.addupdate_scatter(acc, [base_i + 6], ddz * g0, mask=act)
                plsc.addupdate_scatter(acc, [base_i + 7], ddz * g1, mask=act)

            def _edge2(jj2, c2):
                j = jj2 * 2
                _one(j)
                _one(j + 1)
                return c2

            lax.fori_loop(0, (m + 1) // 2, _edge2, 0)
            return c1

        lax.fori_loop(0, nchunks, _chunk, 0)
        return c0

    lax.fori_loop(0, _SB, _superblock, 0)

    wv = []
    for l in range(_NO):
        for n in range(_NMAX):
            for o in range(_NO):
                j = 18 + l * 8 + n * 4 + o
                wv.append(plsc.load_gather(
                    consts_v, [jnp.full((_LANES,), j, jnp.int32)]))

    def _atoms(av, c):
        base = av * (_LANES * _ASTRIDE)
        bvec = []
        for ch in range(8):
            bvec.append(plsc.load_gather(acc, [base + iota * _ASTRIDE + ch]))
        for o in range(_NO):
            ro = None
            for l in range(_NO):
                tl = (wv[(l * 2) * 4 + o] * bvec[l * 2]
                      + wv[(l * 2 + 1) * 4 + o] * bvec[l * 2 + 1])
                ro = tl * tl if ro is None else ro + tl * tl
            plsc.store_scatter(rho_v, [av * (_LANES * 4) + iota * 4 + o], ro)
        return c

    lax.fori_loop(0, _A_TILE // _LANES, _atoms, 0)

    pltpu.sync_copy(rho_v, out_h.at[pl.ds(pl.multiple_of(wid * _RHO_W, 8),
                                          _RHO_W)])


def kernel(disp, iidx, jidx, sorted_numbers, alpha, rs, species_params,
           orbital_params):
    disp = disp.astype(jnp.float32)
    iidx = iidx.astype(jnp.int32)
    jidx = jidx.astype(jnp.int32)
    sorted_numbers = sorted_numbers.astype(jnp.int32)

    ge = lax.shift_right_logical(iidx, 2)
    counts = jnp.zeros((_NGROUP,), jnp.int32).at[ge].add(
        1, mode="drop", indices_are_sorted=True)
    s4 = jnp.concatenate([jnp.zeros((1,), jnp.int32),
                          jnp.cumsum(counts, dtype=jnp.int32)])
    s4 = jnp.concatenate(
        [s4, jnp.full((_S4_W - s4.shape[0],), _E, jnp.int32)])

    oidx = jnp.array([0, 1, 1, 1], dtype=jnp.int32)
    wln = orbital_params[0].astype(jnp.float32)[oidx]
    consts = jnp.concatenate([
        alpha.astype(jnp.float32).reshape(-1),
        rs.astype(jnp.float32).reshape(-1),
        species_params.astype(jnp.float32).reshape(-1),
        wln.reshape(-1),
        jnp.zeros((14,), jnp.float32),
    ])

    mesh = plsc.VectorSubcoreMesh(core_axis_name="c", subcore_axis_name="s")
    call = functools.partial(
        pl.kernel,
        mesh=mesh,
        compiler_params=pltpu.CompilerParams(needs_layout_passes=False),
        out_type=jax.ShapeDtypeStruct((_NA_PAD * _NO,), jnp.float32),
        scratch_types=[
            pltpu.VMEM((_NA,), jnp.int32),
            pltpu.VMEM((64,), jnp.float32),
            pltpu.VMEM((_S4_W,), jnp.int32),
            pltpu.VMEM((_B,), jnp.float32),
            pltpu.VMEM((_B,), jnp.float32),
            pltpu.VMEM((_B,), jnp.float32),
            pltpu.VMEM((_B,), jnp.int32),
            pltpu.VMEM((_B,), jnp.int32),
            pltpu.VMEM((_ACC_W,), jnp.float32),
            pltpu.VMEM((_RHO_W,), jnp.float32),
        ],
    )(_sc_body)

    out = call(disp[:, 0], disp[:, 1], disp[:, 2], iidx, jidx,
               sorted_numbers, consts, s4)
    rho = out.reshape(_NA_PAD, _NO)[:_NA]
    return rho[None]
